# baseline (speedup 1.0000x reference)
import functools

import jax
import jax.numpy as jnp
from jax import lax
from jax.experimental import pallas as pl
from jax.experimental.pallas import tpu as pltpu
from jax.experimental.pallas import tpu_sc as plsc

B, L = 1024, 200
N = B * L
WD, CD = 32, 16
OD = WD + CD
PACK = 128 // WD
WORD_PACKED_ROWS = (1000000 * WD) // 128

_info = plsc.get_sparse_core_info()
NC, NS = _info.num_cores, _info.num_subcores
NW = NC * NS
ROWS_PER_W = N // NW
BPW = B // NW
_LG = 13


def _body(widxT_hbm, cidx_hbm, wtab_hbm, ctab_hbm, out_hbm,
          widxT_v, wraw_v, widxq_v, cidx_v, ctab_v, wbuf, obuf, sem):
    wid = lax.axis_index("s") * NC + lax.axis_index("c")
    base = wid * ROWS_PER_W
    colgrp = (wid // 4) * 128
    colbase = (wid % 4) * BPW

    pltpu.sync_copy(widxT_hbm.at[:, pl.ds(colgrp, 128)],
                    widxT_v.at[pl.ds(0, L)])
    pltpu.sync_copy(cidx_hbm.at[pl.ds(base, ROWS_PER_W)], cidx_v)
    pltpu.sync_copy(ctab_hbm, ctab_v)

    lanes = lax.iota(jnp.int32, 16)
    zero16 = lanes * 0

    def column(b_rel, carry):
        col = zero16 + (colbase + b_rel)
        p0 = b_rel * L
        for j in range(_LG):
            l0 = min(16 * j, L - 16)
            wv = plsc.load_gather(widxT_v, [lanes + l0, col])
            wraw_v[pl.ds(l0, 16)] = wv
            widxq_v[pl.ds(l0, 16)] = lax.shift_right_logical(wv, 2)
        pltpu.async_copy(wtab_hbm.at[widxq_v.at[pl.ds(0, L)]],
                         wbuf, sem).wait()
        for j in range(_LG):
            l0 = min(16 * j, L - 16)
            rows = lanes + l0
            wv = wraw_v[pl.ds(l0, 16)]
            rv = jnp.left_shift(jnp.bitwise_and(wv, PACK - 1), 5)
            cv = cidx_v[pl.ds(p0 + l0, 16)]
            cr = lax.shift_right_logical(cv, 3)
            cc = jnp.left_shift(jnp.bitwise_and(cv, 7), 4)
            for f in range(WD):
                vals = plsc.load_gather(wbuf, [rows, rv + f])
                plsc.store_scatter(obuf, [rows, zero16 + f], vals)
            for f in range(CD):
                vals = plsc.load_gather(ctab_v, [cr, cc + f])
                plsc.store_scatter(obuf, [rows, zero16 + (WD + f)], vals)
        pltpu.sync_copy(obuf, out_hbm.at[pl.ds(base + p0, L)])
        return carry

    lax.fori_loop(0, BPW, column, 0)


@jax.jit
def _run(widxT, cidx, wtab, ctab):
    mesh = plsc.VectorSubcoreMesh(core_axis_name="c", subcore_axis_name="s")
    f = functools.partial(
        pl.kernel,
        mesh=mesh,
        out_type=jax.ShapeDtypeStruct((N, OD), jnp.float32),
        compiler_params=pltpu.CompilerParams(needs_layout_passes=False),
        scratch_types=[
            pltpu.VMEM((208, 128), jnp.int32),
            pltpu.VMEM((208,), jnp.int32),
            pltpu.VMEM((208,), jnp.int32),
            pltpu.VMEM((ROWS_PER_W,), jnp.int32),
            pltpu.VMEM((125, 128), jnp.float32),
            pltpu.VMEM((L, 128), jnp.float32),
            pltpu.VMEM((L, OD), jnp.float32),
            pltpu.SemaphoreType.DMA,
        ],
    )(_body)
    return f(widxT, cidx, wtab, ctab)


def kernel(batchInput, batchChar_input, batchChar_offsets, wordEmb, charEmb):
    del batchChar_offsets
    widxT = batchInput.T.astype(jnp.int32)
    cidx = batchChar_input.astype(jnp.int32)
    wtab = jnp.reshape(wordEmb, (WORD_PACKED_ROWS, 128))
    ctab = jnp.reshape(charEmb, (125, 128))
    out = _run(widxT, cidx, wtab, ctab)
    return out.reshape(B, L, OD)


# speedup vs baseline: 1.3885x; 1.3885x over previous
import functools

import jax
import jax.numpy as jnp
from jax import lax
from jax.experimental import pallas as pl
from jax.experimental.pallas import tpu as pltpu
from jax.experimental.pallas import tpu_sc as plsc

B, L = 1024, 200
N = B * L
V = 1000000
WD, CD = 32, 16
OD = WD + CD

_info = plsc.get_sparse_core_info()
NC, NS = _info.num_cores, _info.num_subcores
NW = NC * NS

FCH = 320
NFCH = V // FCH
FPW = -(-NFCH // NW)
FPAIR = -(-FPW // 2)

ROWS_PER_W = N // NW
BPW = B // NW
CHUNK = 640
NCHUNK = ROWS_PER_W // CHUNK
_LGRP = L // 16 + (1 if L % 16 else 0)


def _flatten_body(wt_hbm, out_hbm, vbuf0, vbuf1, vflat0, vflat1,
                  isem0, isem1, osem0, osem1):
    wid = lax.axis_index("s") * NC + lax.axis_index("c")

    vbufs = (vbuf0, vbuf1)
    vflats = (vflat0, vflat1)
    isems = (isem0, isem1)
    osems = (osem0, osem1)

    def start_in(cid, par):
        @pl.when(cid < NFCH)
        def _():
            pltpu.async_copy(wt_hbm.at[pl.ds(cid * FCH, FCH)],
                             vbufs[par], isems[par])

    def pair(g, carry):
        for par in (0, 1):
            i = g * 2 + par
            cid = i * NW + wid

            @pl.when(cid < NFCH)
            def _():
                vb, vf = vbufs[par], vflats[par]
                pltpu.make_async_copy(wt_hbm.at[pl.ds(cid * FCH, FCH)],
                                      vb, isems[par]).wait()
                @pl.when(cid >= 2 * NW)
                def _():
                    pltpu.make_async_copy(
                        vf, out_hbm.at[pl.ds((cid - 2 * NW) * FCH * WD,
                                             FCH * WD)], osems[par]).wait()
                for r in range(FCH):
                    vf[pl.ds(r * WD, 16)] = vb[r, pl.ds(0, 16)]
                    vf[pl.ds(r * WD + 16, 16)] = vb[r, pl.ds(16, 16)]
                pltpu.async_copy(
                    vf, out_hbm.at[pl.ds(cid * FCH * WD, FCH * WD)],
                    osems[par])
                start_in(cid + 2 * NW, par)
        return carry

    start_in(wid, 0)
    start_in(NW + wid, 1)
    lax.fori_loop(0, FPAIR, pair, 0)
    for par in (0, 1):
        cid = (FPAIR * 2 - 2 + par) * NW + wid

        @pl.when(cid < NFCH)
        def _():
            pltpu.make_async_copy(
                vflats[par],
                out_hbm.at[pl.ds(cid * FCH * WD, FCH * WD)],
                osems[par]).wait()


def _gather_body(widxT_hbm, cidx_hbm, wtab_hbm, ctab_hbm, out_hbm,
                 widxT_v, widx_v, cidx_v, wbuf, cbuf, sem):
    wid = lax.axis_index("s") * NC + lax.axis_index("c")
    base = wid * ROWS_PER_W
    b0 = wid * BPW

    pltpu.sync_copy(widxT_hbm.at[:, pl.ds(b0, BPW)], widxT_v.at[pl.ds(0, L)])
    lanes = lax.iota(jnp.int32, 16)
    zero16 = lanes * 0
    for b_rel in range(BPW):
        col = zero16 + b_rel
        for j in range(_LGRP):
            vals = plsc.load_gather(widxT_v, [lanes + j * 16, col])
            widx_v[pl.ds(b_rel * L + j * 16, 16)] = vals

    def chunk_body(i, carry):
        off = base + i * CHUNK
        pltpu.sync_copy(cidx_hbm.at[pl.ds(off, CHUNK)], cidx_v)
        pltpu.async_copy(wtab_hbm.at[widx_v.at[pl.ds(i * CHUNK, CHUNK)]],
                         wbuf, sem).wait()
        pltpu.async_copy(ctab_hbm.at[cidx_v], cbuf, sem).wait()
        pltpu.sync_copy(wbuf, out_hbm.at[pl.ds(off, CHUNK), pl.ds(0, WD)])
        pltpu.sync_copy(cbuf, out_hbm.at[pl.ds(off, CHUNK), pl.ds(WD, CD)])
        return carry

    lax.fori_loop(0, NCHUNK, chunk_body, 0)


_MESH = dict(core_axis_name="c", subcore_axis_name="s")


@jax.jit
def _run(widxT, cidx, wtab_tiled, ctab):
    flat = functools.partial(
        pl.kernel,
        mesh=plsc.VectorSubcoreMesh(**_MESH),
        out_type=jax.ShapeDtypeStruct((V * WD,), jnp.float32),
        compiler_params=pltpu.CompilerParams(needs_layout_passes=False),
        scratch_types=[
            pltpu.VMEM((FCH, WD), jnp.float32),
            pltpu.VMEM((FCH, WD), jnp.float32),
            pltpu.VMEM((FCH * WD,), jnp.float32),
            pltpu.VMEM((FCH * WD,), jnp.float32),
            pltpu.SemaphoreType.DMA,
            pltpu.SemaphoreType.DMA,
            pltpu.SemaphoreType.DMA,
            pltpu.SemaphoreType.DMA,
        ],
    )(_flatten_body)
    wtab_flat = flat(wtab_tiled)
    wtab = jnp.reshape(wtab_flat, (V, WD))

    g = functools.partial(
        pl.kernel,
        mesh=plsc.VectorSubcoreMesh(**_MESH),
        out_type=jax.ShapeDtypeStruct((N, OD), jnp.float32),
        compiler_params=pltpu.CompilerParams(use_tc_tiling_on_sc=False,
                                             needs_layout_passes=False),
        scratch_types=[
            pltpu.VMEM((_LGRP * 16, BPW), jnp.int32),
            pltpu.VMEM((ROWS_PER_W + 16,), jnp.int32),
            pltpu.VMEM((CHUNK,), jnp.int32),
            pltpu.VMEM((CHUNK, WD), jnp.float32),
            pltpu.VMEM((CHUNK, CD), jnp.float32),
            pltpu.SemaphoreType.DMA,
        ],
    )(_gather_body)
    return g(widxT, cidx, wtab, ctab)


def kernel(batchInput, batchChar_input, batchChar_offsets, wordEmb, charEmb):
    del batchChar_offsets
    widxT = batchInput.T.astype(jnp.int32)
    cidx = batchChar_input.astype(jnp.int32)
    out = _run(widxT, cidx, wordEmb, charEmb)
    return out.reshape(B, L, OD)
